# baseline (speedup 1.0000x reference)
import functools

import jax
import jax.numpy as jnp
from jax import lax
from jax.experimental import pallas as pl
from jax.experimental.pallas import tpu as pltpu
from jax.experimental.pallas import tpu_sc as plsc


def _matmul_body(w_ref, xt_ref, o_ref):
    o_ref[...] = jnp.dot(w_ref[...], xt_ref[...],
                         preferred_element_type=jnp.float32)


def _gate_body(x_ref, a_ref, o_ref):
    o_ref[...] = x_ref[...] * jax.nn.sigmoid(a_ref[...])


def _make_sc_gather(K, N, NP, G):
    f32 = jnp.float32
    PT = NP // 32
    NG = K // G
    GSZ = G * PT
    U = 8

    mesh = plsc.VectorSubcoreMesh(core_axis_name="c", subcore_axis_name="s")

    @functools.partial(
        pl.kernel,
        out_type=jax.ShapeDtypeStruct((NP,), f32),
        mesh=mesh,
        compiler_params=pltpu.CompilerParams(needs_layout_passes=False),
        scratch_types=[
            pltpu.VMEM((GSZ,), jnp.int32),
            pltpu.VMEM((GSZ,), jnp.int32),
            pltpu.VMEM((GSZ,), f32),
            pltpu.VMEM((GSZ,), f32),
            pltpu.VMEM((PT,), f32),
            pltpu.SemaphoreType.DMA,
            pltpu.SemaphoreType.DMA,
        ],
    )
    def sc_gather(gidx_hbm, yt_hbm, s_hbm,
                  idx0, idx1, gb0, gb1, acc, sem_i, sem_g):
        c = lax.axis_index("c")
        s = lax.axis_index("s")
        w = s * 2 + c
        base = w * PT
        blk = w * (K * PT)
        idxb = (idx0, idx1)
        gbufs = (gb0, gb1)

        pltpu.async_copy(gidx_hbm.at[pl.ds(blk, GSZ)], idx0, sem_i).wait()
        gathers = [None] * NG
        gathers[0] = pltpu.async_copy(yt_hbm.at[idx0], gb0, sem_g)
        idx_pending = None
        if NG > 1:
            idx_pending = pltpu.async_copy(
                gidx_hbm.at[pl.ds(blk + GSZ, GSZ)], idx1, sem_i)

        for g in range(NG):
            gathers[g].wait()
            if g + 1 < NG:
                idx_pending.wait()
                gathers[g + 1] = pltpu.async_copy(
                    yt_hbm.at[idxb[(g + 1) % 2]], gbufs[(g + 1) % 2], sem_g)
                if g + 2 < NG:
                    idx_pending = pltpu.async_copy(
                        gidx_hbm.at[pl.ds(blk + (g + 2) * GSZ, GSZ)],
                        idxb[g % 2], sem_i)
            gb = gbufs[g % 2]

            def accum(j, carry, gb=gb, first=(g == 0)):
                o = j * (16 * U)
                for u in range(U):
                    oo = o + u * 16
                    v = gb[pl.ds(oo, 16)]
                    for r in range(1, G):
                        v = v + gb[pl.ds(r * PT + oo, 16)]
                    if first:
                        acc[pl.ds(oo, 16)] = v
                    else:
                        acc[pl.ds(oo, 16)] = acc[pl.ds(oo, 16)] + v
                return carry

            lax.fori_loop(0, PT // (16 * U), accum, 0)

        pltpu.sync_copy(acc, s_hbm.at[pl.ds(base, PT)])

    return sc_gather


def kernel(x, neighbor_map, W):
    N, C = x.shape
    K = neighbor_map.shape[1]
    f32 = jnp.float32

    BC = 4096
    NP = ((N + BC - 1) // BC) * BC

    Wk = W.reshape(K, C)
    xT = x.T
    PT = NP // 32
    gidx = (neighbor_map.T.astype(jnp.int32)
            + jnp.arange(K, dtype=jnp.int32)[:, None] * N)
    gidx = jnp.pad(gidx, ((0, 0), (0, NP - N)))
    gidx = gidx.reshape(K, 32, PT).transpose(1, 0, 2).reshape(-1)

    BA = 2048
    ga = (N + BA - 1) // BA
    yt = pl.pallas_call(
        _matmul_body,
        grid=(ga,),
        in_specs=[pl.BlockSpec((K, C), lambda i: (0, 0)),
                  pl.BlockSpec((C, BA), lambda i: (0, i))],
        out_specs=pl.BlockSpec((K, BA), lambda i: (0, i)),
        out_shape=jax.ShapeDtypeStruct((K, N), f32),
    )(Wk, xT)

    sc = _make_sc_gather(K, N, NP, G=9)
    s = sc(gidx, yt.reshape(-1))

    st = s[:N].reshape(N, 1)
    BB = 2048
    gb = (N + BB - 1) // BB
    out = pl.pallas_call(
        _gate_body,
        grid=(gb,),
        in_specs=[pl.BlockSpec((BB, C), lambda i: (i, 0)),
                  pl.BlockSpec((BB, 1), lambda i: (i, 0))],
        out_specs=pl.BlockSpec((BB, C), lambda i: (i, 0)),
        out_shape=jax.ShapeDtypeStruct((N, C), f32),
    )(x, st)
    return out


# speedup vs baseline: 1.6005x; 1.6005x over previous
import functools

import jax
import jax.numpy as jnp
from jax import lax
from jax.experimental import pallas as pl
from jax.experimental.pallas import tpu as pltpu
from jax.experimental.pallas import tpu_sc as plsc


def _matmul_body(w_ref, xt_ref, o_ref):
    o_ref[...] = jnp.dot(w_ref[...], xt_ref[...],
                         preferred_element_type=jnp.float32)


def _gate_body(x_ref, a_ref, b_ref, o_ref):
    o_ref[...] = x_ref[...] * jax.nn.sigmoid(a_ref[...] + b_ref[...])


def _make_sc_gather(K, N, NP):
    f32 = jnp.float32
    PT = NP // 16
    ROWS = 14
    U = 8

    mesh = plsc.VectorSubcoreMesh(core_axis_name="c", subcore_axis_name="s")

    @functools.partial(
        pl.kernel,
        out_type=[jax.ShapeDtypeStruct((NP,), f32),
                  jax.ShapeDtypeStruct((NP,), f32)],
        mesh=mesh,
        compiler_params=pltpu.CompilerParams(needs_layout_passes=False),
        scratch_types=[
            pltpu.VMEM((PT,), jnp.int32),
            pltpu.VMEM((PT,), jnp.int32),
            pltpu.VMEM((PT,), f32),
            pltpu.VMEM((PT,), f32),
            pltpu.VMEM((PT,), f32),
            pltpu.VMEM_SHARED((ROWS * N,), f32),
            pltpu.SemaphoreType.DMA,
            pltpu.SemaphoreType.DMA,
        ],
    )
    def sc_gather(gidx_hbm, yt_hbm, s0_hbm, s1_hbm,
                  idx0, idx1, gb0, gb1, acc, ytsh, sem_i, sem_g):
        c = lax.axis_index("c")
        s = lax.axis_index("s")

        n_real = 14 - c

        @pl.when(s < n_real)
        def _stage():
            krow = c * ROWS + s
            nfull = N // PT
            for ci in range(nfull):
                pltpu.sync_copy(yt_hbm.at[pl.ds(krow * N + ci * PT, PT)], gb1)
                pltpu.sync_copy(gb1, ytsh.at[pl.ds(s * N + ci * PT, PT)])
            rem = N - nfull * PT
            if rem:
                pltpu.sync_copy(
                    yt_hbm.at[pl.ds(krow * N + nfull * PT, rem)],
                    gb1.at[pl.ds(0, rem)])
                pltpu.sync_copy(
                    gb1.at[pl.ds(0, rem)],
                    ytsh.at[pl.ds(s * N + nfull * PT, rem)])

        @pl.when((c == 1) & (s == ROWS - 1))
        def _zero_pad_row():
            def zv(j, carry):
                o = j * (16 * U)
                for u in range(U):
                    gb0[pl.ds(o + u * 16, 16)] = jnp.zeros((16,), f32)
                return carry

            lax.fori_loop(0, PT // (16 * U), zv, 0)
            for ci in range(N // PT):
                pltpu.sync_copy(
                    gb0, ytsh.at[pl.ds((ROWS - 1) * N + ci * PT, PT)])
            rem = N - (N // PT) * PT
            if rem:
                pltpu.sync_copy(
                    gb0.at[pl.ds(0, rem)],
                    ytsh.at[pl.ds((ROWS - 1) * N + (N // PT) * PT, rem)])

        plsc.subcore_barrier()

        blk = (c * 16 + s) * (ROWS * PT)
        idxb = (idx0, idx1)
        gbufs = (gb0, gb1)

        pltpu.async_copy(gidx_hbm.at[pl.ds(blk, PT)], idx0, sem_i).wait()
        gathers = [None] * ROWS
        gathers[0] = pltpu.async_copy(ytsh.at[idx0], gb0, sem_g)
        idx_pending = pltpu.async_copy(
            gidx_hbm.at[pl.ds(blk + PT, PT)], idx1, sem_i)

        for g in range(ROWS):
            gathers[g].wait()
            if g + 1 < ROWS:
                idx_pending.wait()
                gathers[g + 1] = pltpu.async_copy(
                    ytsh.at[idxb[(g + 1) % 2]], gbufs[(g + 1) % 2], sem_g)
                if g + 2 < ROWS:
                    idx_pending = pltpu.async_copy(
                        gidx_hbm.at[pl.ds(blk + (g + 2) * PT, PT)],
                        idxb[g % 2], sem_i)
            gb = gbufs[g % 2]

            def accum(j, carry, gb=gb, first=(g == 0)):
                o = j * (16 * U)
                for u in range(U):
                    oo = o + u * 16
                    if first:
                        acc[pl.ds(oo, 16)] = gb[pl.ds(oo, 16)]
                    else:
                        acc[pl.ds(oo, 16)] = acc[pl.ds(oo, 16)] + gb[pl.ds(oo, 16)]
                return carry

            lax.fori_loop(0, PT // (16 * U), accum, 0)

        @pl.when(c == 0)
        def _w0():
            pltpu.sync_copy(acc, s0_hbm.at[pl.ds(s * PT, PT)])

        @pl.when(c == 1)
        def _w1():
            pltpu.sync_copy(acc, s1_hbm.at[pl.ds(s * PT, PT)])

    return sc_gather


def kernel(x, neighbor_map, W):
    N, C = x.shape
    K = neighbor_map.shape[1]
    f32 = jnp.float32

    BC = 4096
    NP = ((N + BC - 1) // BC) * BC

    Wk = W.reshape(K, C)
    xT = x.T
    PT = NP // 16
    offs = jnp.concatenate([
        jnp.arange(14, dtype=jnp.int32),
        jnp.arange(13, dtype=jnp.int32),
        jnp.full((1,), 13, dtype=jnp.int32)]) * N
    nmT28 = jnp.concatenate([
        neighbor_map.T.astype(jnp.int32),
        jnp.zeros((1, N), jnp.int32)], axis=0)
    gidx = jnp.pad(nmT28 + offs[:, None], ((0, 0), (0, NP - N)))
    gidx = gidx.reshape(2, 14, 16, PT).transpose(0, 2, 1, 3).reshape(-1)

    BA = 2048
    ga = (N + BA - 1) // BA
    yt = pl.pallas_call(
        _matmul_body,
        grid=(ga,),
        in_specs=[pl.BlockSpec((K, C), lambda i: (0, 0)),
                  pl.BlockSpec((C, BA), lambda i: (0, i))],
        out_specs=pl.BlockSpec((K, BA), lambda i: (0, i)),
        out_shape=jax.ShapeDtypeStruct((K, N), f32),
    )(Wk, xT)

    sc = _make_sc_gather(K, N, NP)
    s0, s1 = sc(gidx, yt.reshape(-1))

    s0t = s0[:N].reshape(N, 1)
    s1t = s1[:N].reshape(N, 1)
    BB = 2048
    gb = (N + BB - 1) // BB
    out = pl.pallas_call(
        _gate_body,
        grid=(gb,),
        in_specs=[pl.BlockSpec((BB, C), lambda i: (i, 0)),
                  pl.BlockSpec((BB, 1), lambda i: (i, 0)),
                  pl.BlockSpec((BB, 1), lambda i: (i, 0))],
        out_specs=pl.BlockSpec((BB, C), lambda i: (i, 0)),
        out_shape=jax.ShapeDtypeStruct((N, C), f32),
    )(x, s0t, s1t)
    return out


# speedup vs baseline: 2.0905x; 1.3062x over previous
import functools

import jax
import jax.numpy as jnp
from jax import lax
from jax.experimental import pallas as pl
from jax.experimental.pallas import tpu as pltpu
from jax.experimental.pallas import tpu_sc as plsc


def _matmul_body(w_ref, xt_ref, o_ref):
    o_ref[...] = jnp.dot(w_ref[...], xt_ref[...],
                         preferred_element_type=jnp.float32)


def _gate_body(x_ref, a_ref, b_ref, o_ref):
    att = jax.nn.sigmoid(a_ref[...] + b_ref[...])
    o_ref[...] = x_ref[...] * att.T


def _make_sc_gather(K, N, NP):
    f32 = jnp.float32
    PT = NP // 16
    ROWS = 14
    U = 8

    mesh = plsc.VectorSubcoreMesh(core_axis_name="c", subcore_axis_name="s")

    @functools.partial(
        pl.kernel,
        out_type=[jax.ShapeDtypeStruct((NP,), f32),
                  jax.ShapeDtypeStruct((NP,), f32)],
        mesh=mesh,
        compiler_params=pltpu.CompilerParams(needs_layout_passes=False),
        scratch_types=[
            pltpu.VMEM((PT,), jnp.int32),
            pltpu.VMEM((PT,), jnp.int32),
            pltpu.VMEM((PT,), f32),
            pltpu.VMEM((PT,), f32),
            pltpu.VMEM((PT,), f32),
            pltpu.VMEM_SHARED((ROWS * N,), f32),
            pltpu.SemaphoreType.DMA,
            pltpu.SemaphoreType.DMA,
        ],
    )
    def sc_gather(gidx_hbm, yt_hbm, s0_hbm, s1_hbm,
                  idx0, idx1, gb0, gb1, acc, ytsh, sem_i, sem_g):
        c = lax.axis_index("c")
        s = lax.axis_index("s")

        n_real = 14 - c

        @pl.when(s < n_real)
        def _stage():
            krow = c * ROWS + s
            nfull = N // PT
            for ci in range(nfull):
                pltpu.sync_copy(yt_hbm.at[pl.ds(krow * N + ci * PT, PT)], gb1)
                pltpu.sync_copy(gb1, ytsh.at[pl.ds(s * N + ci * PT, PT)])
            rem = N - nfull * PT
            if rem:
                pltpu.sync_copy(
                    yt_hbm.at[pl.ds(krow * N + nfull * PT, rem)],
                    gb1.at[pl.ds(0, rem)])
                pltpu.sync_copy(
                    gb1.at[pl.ds(0, rem)],
                    ytsh.at[pl.ds(s * N + nfull * PT, rem)])

        @pl.when((c == 1) & (s == ROWS - 1))
        def _zero_pad_row():
            def zv(j, carry):
                o = j * (16 * U)
                for u in range(U):
                    gb0[pl.ds(o + u * 16, 16)] = jnp.zeros((16,), f32)
                return carry

            lax.fori_loop(0, PT // (16 * U), zv, 0)
            for ci in range(N // PT):
                pltpu.sync_copy(
                    gb0, ytsh.at[pl.ds((ROWS - 1) * N + ci * PT, PT)])
            rem = N - (N // PT) * PT
            if rem:
                pltpu.sync_copy(
                    gb0.at[pl.ds(0, rem)],
                    ytsh.at[pl.ds((ROWS - 1) * N + (N // PT) * PT, rem)])

        plsc.subcore_barrier()

        blk = (c * 16 + s) * (ROWS * PT)
        idxb = (idx0, idx1)
        gbufs = (gb0, gb1)

        pltpu.async_copy(gidx_hbm.at[pl.ds(blk, PT)], idx0, sem_i).wait()
        gathers = [None] * ROWS
        gathers[0] = pltpu.async_copy(ytsh.at[idx0], gb0, sem_g)
        idx_pending = pltpu.async_copy(
            gidx_hbm.at[pl.ds(blk + PT, PT)], idx1, sem_i)

        for g in range(ROWS):
            gathers[g].wait()
            if g + 1 < ROWS:
                idx_pending.wait()
                gathers[g + 1] = pltpu.async_copy(
                    ytsh.at[idxb[(g + 1) % 2]], gbufs[(g + 1) % 2], sem_g)
                if g + 2 < ROWS:
                    idx_pending = pltpu.async_copy(
                        gidx_hbm.at[pl.ds(blk + (g + 2) * PT, PT)],
                        idxb[g % 2], sem_i)
            gb = gbufs[g % 2]

            def accum(j, carry, gb=gb, first=(g == 0)):
                o = j * (16 * U)
                for u in range(U):
                    oo = o + u * 16
                    if first:
                        acc[pl.ds(oo, 16)] = gb[pl.ds(oo, 16)]
                    else:
                        acc[pl.ds(oo, 16)] = acc[pl.ds(oo, 16)] + gb[pl.ds(oo, 16)]
                return carry

            lax.fori_loop(0, PT // (16 * U), accum, 0)

        @pl.when(c == 0)
        def _w0():
            pltpu.sync_copy(acc, s0_hbm.at[pl.ds(s * PT, PT)])

        @pl.when(c == 1)
        def _w1():
            pltpu.sync_copy(acc, s1_hbm.at[pl.ds(s * PT, PT)])

    return sc_gather


def kernel(x, neighbor_map, W):
    N, C = x.shape
    K = neighbor_map.shape[1]
    f32 = jnp.float32

    BC = 4096
    NP = ((N + BC - 1) // BC) * BC

    Wk = W.reshape(K, C)
    xT = x.T
    PT = NP // 16
    offs = jnp.concatenate([
        jnp.arange(14, dtype=jnp.int32),
        jnp.arange(13, dtype=jnp.int32),
        jnp.full((1,), 13, dtype=jnp.int32)]) * N
    nmT28 = jnp.concatenate([
        neighbor_map.T.astype(jnp.int32),
        jnp.zeros((1, N), jnp.int32)], axis=0)
    gidx = jnp.pad(nmT28 + offs[:, None], ((0, 0), (0, NP - N)))
    gidx = gidx.reshape(2, 14, 16, PT).transpose(0, 2, 1, 3).reshape(-1)

    BA = 2048
    ga = (N + BA - 1) // BA
    yt = pl.pallas_call(
        _matmul_body,
        grid=(ga,),
        in_specs=[pl.BlockSpec((K, C), lambda i: (0, 0)),
                  pl.BlockSpec((C, BA), lambda i: (0, i))],
        out_specs=pl.BlockSpec((K, BA), lambda i: (0, i)),
        out_shape=jax.ShapeDtypeStruct((K, N), f32),
    )(Wk, xT)

    sc = _make_sc_gather(K, N, NP)
    s0, s1 = sc(gidx, yt.reshape(-1))

    s0t = s0.reshape(1, NP)
    s1t = s1.reshape(1, NP)
    BB = 2048
    gb = (N + BB - 1) // BB
    out = pl.pallas_call(
        _gate_body,
        grid=(gb,),
        in_specs=[pl.BlockSpec((BB, C), lambda i: (i, 0)),
                  pl.BlockSpec((1, BB), lambda i: (0, i)),
                  pl.BlockSpec((1, BB), lambda i: (0, i))],
        out_specs=pl.BlockSpec((BB, C), lambda i: (i, 0)),
        out_shape=jax.ShapeDtypeStruct((N, C), f32),
    )(x, s0t, s1t)
    return out


# speedup vs baseline: 2.2205x; 1.0622x over previous
import functools

import jax
import jax.numpy as jnp
from jax import lax
from jax.experimental import pallas as pl
from jax.experimental.pallas import tpu as pltpu
from jax.experimental.pallas import tpu_sc as plsc


def _matmul_body(w_ref, xt_ref, o_ref):
    o_ref[...] = jnp.dot(w_ref[...], xt_ref[...],
                         preferred_element_type=jnp.float32)


def _gate_body(x_ref, a_ref, b_ref, o_ref):
    att = jax.nn.sigmoid(a_ref[...] + b_ref[...])
    o_ref[...] = x_ref[...] * att.T


def _make_sc_gather(K, N, NP):
    f32 = jnp.float32
    PT = NP // 16
    ROWS = 14
    U = 8

    mesh = plsc.VectorSubcoreMesh(core_axis_name="c", subcore_axis_name="s")

    @functools.partial(
        pl.kernel,
        out_type=[jax.ShapeDtypeStruct((NP,), f32),
                  jax.ShapeDtypeStruct((NP,), f32)],
        mesh=mesh,
        compiler_params=pltpu.CompilerParams(needs_layout_passes=False),
        scratch_types=[
            pltpu.VMEM((PT,), jnp.int32),
            pltpu.VMEM((PT,), jnp.int32),
            pltpu.VMEM((PT,), f32),
            pltpu.VMEM((PT,), f32),
            pltpu.VMEM((PT,), f32),
            pltpu.VMEM_SHARED((ROWS * N,), f32),
            pltpu.SemaphoreType.DMA,
            pltpu.SemaphoreType.DMA,
        ],
    )
    def sc_gather(gidx_hbm, yt_hbm, s0_hbm, s1_hbm,
                  idx0, idx1, gb0, gb1, acc, ytsh, sem_i, sem_g):
        c = lax.axis_index("c")
        s = lax.axis_index("s")

        n_real = 14 - c

        @pl.when(s < n_real)
        def _stage():
            krow = c * ROWS + s
            nfull = N // PT
            rem = N - nfull * PT
            bufs = (gb0, gb1)
            h = pltpu.async_copy(yt_hbm.at[pl.ds(krow * N, PT)], gb0, sem_i)
            for ci in range(nfull):
                h.wait()
                if ci + 1 < nfull:
                    h = pltpu.async_copy(
                        yt_hbm.at[pl.ds(krow * N + (ci + 1) * PT, PT)],
                        bufs[(ci + 1) % 2], sem_i)
                elif rem:
                    h = pltpu.async_copy(
                        yt_hbm.at[pl.ds(krow * N + nfull * PT, rem)],
                        bufs[(ci + 1) % 2].at[pl.ds(0, rem)], sem_i)
                pltpu.sync_copy(bufs[ci % 2], ytsh.at[pl.ds(s * N + ci * PT, PT)])
            if rem:
                h.wait()
                pltpu.sync_copy(bufs[nfull % 2].at[pl.ds(0, rem)],
                                ytsh.at[pl.ds(s * N + nfull * PT, rem)])

        @pl.when((c == 1) & (s == ROWS - 1))
        def _zero_pad_row():
            def zv(j, carry):
                o = j * (16 * U)
                for u in range(U):
                    gb0[pl.ds(o + u * 16, 16)] = jnp.zeros((16,), f32)
                return carry

            lax.fori_loop(0, PT // (16 * U), zv, 0)
            for ci in range(N // PT):
                pltpu.sync_copy(
                    gb0, ytsh.at[pl.ds((ROWS - 1) * N + ci * PT, PT)])
            rem = N - (N // PT) * PT
            if rem:
                pltpu.sync_copy(
                    gb0.at[pl.ds(0, rem)],
                    ytsh.at[pl.ds((ROWS - 1) * N + (N // PT) * PT, rem)])

        plsc.subcore_barrier()

        rbase = c * ROWS * NP + s * PT
        idxb = (idx0, idx1)
        gbufs = (gb0, gb1)

        pltpu.async_copy(gidx_hbm.at[pl.ds(rbase, PT)], idx0, sem_i).wait()
        gathers = [None] * ROWS
        gathers[0] = pltpu.async_copy(ytsh.at[idx0], gb0, sem_g)
        idx_pending = pltpu.async_copy(
            gidx_hbm.at[pl.ds(rbase + NP, PT)], idx1, sem_i)

        for g in range(ROWS):
            gathers[g].wait()
            if g + 1 < ROWS:
                idx_pending.wait()
                gathers[g + 1] = pltpu.async_copy(
                    ytsh.at[idxb[(g + 1) % 2]], gbufs[(g + 1) % 2], sem_g)
                if g + 2 < ROWS:
                    idx_pending = pltpu.async_copy(
                        gidx_hbm.at[pl.ds(rbase + (g + 2) * NP, PT)],
                        idxb[g % 2], sem_i)
            gb = gbufs[g % 2]

            def accum(j, carry, gb=gb, first=(g == 0)):
                o = j * (16 * U)
                for u in range(U):
                    oo = o + u * 16
                    if first:
                        acc[pl.ds(oo, 16)] = gb[pl.ds(oo, 16)]
                    else:
                        acc[pl.ds(oo, 16)] = acc[pl.ds(oo, 16)] + gb[pl.ds(oo, 16)]
                return carry

            lax.fori_loop(0, PT // (16 * U), accum, 0)

        @pl.when(c == 0)
        def _w0():
            pltpu.sync_copy(acc, s0_hbm.at[pl.ds(s * PT, PT)])

        @pl.when(c == 1)
        def _w1():
            pltpu.sync_copy(acc, s1_hbm.at[pl.ds(s * PT, PT)])

    return sc_gather


def kernel(x, neighbor_map, W):
    N, C = x.shape
    K = neighbor_map.shape[1]
    f32 = jnp.float32

    BC = 4096
    NP = ((N + BC - 1) // BC) * BC

    Wk = W.reshape(K, C)
    xT = x.T
    PT = NP // 16
    offs = jnp.concatenate([
        jnp.arange(14, dtype=jnp.int32),
        jnp.arange(13, dtype=jnp.int32),
        jnp.full((1,), 13, dtype=jnp.int32)]) * N
    nmT28 = jnp.concatenate([
        neighbor_map.T.astype(jnp.int32),
        jnp.zeros((1, N), jnp.int32)], axis=0)
    gidx = jnp.pad(nmT28 + offs[:, None], ((0, 0), (0, NP - N))).reshape(-1)

    BA = 2048
    ga = (N + BA - 1) // BA
    yt = pl.pallas_call(
        _matmul_body,
        grid=(ga,),
        in_specs=[pl.BlockSpec((K, C), lambda i: (0, 0)),
                  pl.BlockSpec((C, BA), lambda i: (0, i))],
        out_specs=pl.BlockSpec((K, BA), lambda i: (0, i)),
        out_shape=jax.ShapeDtypeStruct((K, N), f32),
    )(Wk, xT)

    sc = _make_sc_gather(K, N, NP)
    s0, s1 = sc(gidx, yt.reshape(-1))

    s0t = s0.reshape(1, NP)
    s1t = s1.reshape(1, NP)
    BB = 2048
    gb = (N + BB - 1) // BB
    out = pl.pallas_call(
        _gate_body,
        grid=(gb,),
        in_specs=[pl.BlockSpec((BB, C), lambda i: (i, 0)),
                  pl.BlockSpec((1, BB), lambda i: (0, i)),
                  pl.BlockSpec((1, BB), lambda i: (0, i))],
        out_specs=pl.BlockSpec((BB, C), lambda i: (i, 0)),
        out_shape=jax.ShapeDtypeStruct((N, C), f32),
    )(x, s0t, s1t)
    return out


# speedup vs baseline: 2.8966x; 1.3045x over previous
import functools

import jax
import jax.numpy as jnp
from jax import lax
from jax.experimental import pallas as pl
from jax.experimental.pallas import tpu as pltpu
from jax.experimental.pallas import tpu_sc as plsc


def _matmul_body(w_ref, xt_ref, o_ref):
    o_ref[...] = jnp.dot(w_ref[...], xt_ref[...],
                         preferred_element_type=jnp.float32)


def _gate_body(xt_ref, a_ref, b_ref, o_ref):
    att = jax.nn.sigmoid(a_ref[...] + b_ref[...])
    o_ref[...] = xt_ref[...] * att


def _make_sc_gather(K, N, NP):
    f32 = jnp.float32
    PT = NP // 16
    ROWS = 14
    U = 8

    mesh = plsc.VectorSubcoreMesh(core_axis_name="c", subcore_axis_name="s")

    @functools.partial(
        pl.kernel,
        out_type=[jax.ShapeDtypeStruct((NP,), f32),
                  jax.ShapeDtypeStruct((NP,), f32)],
        mesh=mesh,
        compiler_params=pltpu.CompilerParams(needs_layout_passes=False),
        scratch_types=[
            pltpu.VMEM((PT,), jnp.int32),
            pltpu.VMEM((PT,), jnp.int32),
            pltpu.VMEM((PT,), f32),
            pltpu.VMEM((PT,), f32),
            pltpu.VMEM((PT,), f32),
            pltpu.VMEM_SHARED((ROWS * N,), f32),
            pltpu.SemaphoreType.DMA,
            pltpu.SemaphoreType.DMA,
        ],
    )
    def sc_gather(gidx_hbm, yt_hbm, s0_hbm, s1_hbm,
                  idx0, idx1, gb0, gb1, acc, ytsh, sem_i, sem_g):
        c = lax.axis_index("c")
        s = lax.axis_index("s")

        n_real = 14 - c

        @pl.when(s < n_real)
        def _stage():
            krow = c * ROWS + s
            nfull = N // PT
            rem = N - nfull * PT
            bufs = (gb0, gb1)
            h = pltpu.async_copy(yt_hbm.at[pl.ds(krow * N, PT)], gb0, sem_i)
            for ci in range(nfull):
                h.wait()
                if ci + 1 < nfull:
                    h = pltpu.async_copy(
                        yt_hbm.at[pl.ds(krow * N + (ci + 1) * PT, PT)],
                        bufs[(ci + 1) % 2], sem_i)
                elif rem:
                    h = pltpu.async_copy(
                        yt_hbm.at[pl.ds(krow * N + nfull * PT, rem)],
                        bufs[(ci + 1) % 2].at[pl.ds(0, rem)], sem_i)
                pltpu.sync_copy(bufs[ci % 2], ytsh.at[pl.ds(s * N + ci * PT, PT)])
            if rem:
                h.wait()
                pltpu.sync_copy(bufs[nfull % 2].at[pl.ds(0, rem)],
                                ytsh.at[pl.ds(s * N + nfull * PT, rem)])

        @pl.when((c == 1) & (s == ROWS - 1))
        def _zero_pad_row():
            def zv(j, carry):
                o = j * (16 * U)
                for u in range(U):
                    gb0[pl.ds(o + u * 16, 16)] = jnp.zeros((16,), f32)
                return carry

            lax.fori_loop(0, PT // (16 * U), zv, 0)
            for ci in range(N // PT):
                pltpu.sync_copy(
                    gb0, ytsh.at[pl.ds((ROWS - 1) * N + ci * PT, PT)])
            rem = N - (N // PT) * PT
            if rem:
                pltpu.sync_copy(
                    gb0.at[pl.ds(0, rem)],
                    ytsh.at[pl.ds((ROWS - 1) * N + (N // PT) * PT, rem)])

        plsc.subcore_barrier()

        rbase = c * ROWS * NP + s * PT
        idxb = (idx0, idx1)
        gbufs = (gb0, gb1)

        pltpu.async_copy(gidx_hbm.at[pl.ds(rbase, PT)], idx0, sem_i).wait()
        gathers = [None] * ROWS
        gathers[0] = pltpu.async_copy(ytsh.at[idx0], gb0, sem_g)
        idx_pending = pltpu.async_copy(
            gidx_hbm.at[pl.ds(rbase + NP, PT)], idx1, sem_i)

        for g in range(ROWS):
            gathers[g].wait()
            if g + 1 < ROWS:
                idx_pending.wait()
                gathers[g + 1] = pltpu.async_copy(
                    ytsh.at[idxb[(g + 1) % 2]], gbufs[(g + 1) % 2], sem_g)
                if g + 2 < ROWS:
                    idx_pending = pltpu.async_copy(
                        gidx_hbm.at[pl.ds(rbase + (g + 2) * NP, PT)],
                        idxb[g % 2], sem_i)
            gb = gbufs[g % 2]

            def accum(j, carry, gb=gb, first=(g == 0)):
                o = j * (16 * U)
                for u in range(U):
                    oo = o + u * 16
                    if first:
                        acc[pl.ds(oo, 16)] = gb[pl.ds(oo, 16)]
                    else:
                        acc[pl.ds(oo, 16)] = acc[pl.ds(oo, 16)] + gb[pl.ds(oo, 16)]
                return carry

            lax.fori_loop(0, PT // (16 * U), accum, 0)

        @pl.when(c == 0)
        def _w0():
            pltpu.sync_copy(acc, s0_hbm.at[pl.ds(s * PT, PT)])

        @pl.when(c == 1)
        def _w1():
            pltpu.sync_copy(acc, s1_hbm.at[pl.ds(s * PT, PT)])

    return sc_gather


def kernel(x, neighbor_map, W):
    N, C = x.shape
    K = neighbor_map.shape[1]
    f32 = jnp.float32

    BC = 4096
    NP = ((N + BC - 1) // BC) * BC

    Wk = W.reshape(K, C)
    xT = x.T
    PT = NP // 16
    offs = jnp.concatenate([
        jnp.arange(14, dtype=jnp.int32),
        jnp.arange(13, dtype=jnp.int32),
        jnp.full((1,), 13, dtype=jnp.int32)]) * N
    nmT28 = jnp.concatenate([
        neighbor_map.T.astype(jnp.int32),
        jnp.zeros((1, N), jnp.int32)], axis=0)
    gidx = jnp.pad(nmT28 + offs[:, None], ((0, 0), (0, NP - N))).reshape(-1)

    BA = 2048
    ga = (N + BA - 1) // BA
    yt = pl.pallas_call(
        _matmul_body,
        grid=(ga,),
        in_specs=[pl.BlockSpec((K, C), lambda i: (0, 0)),
                  pl.BlockSpec((C, BA), lambda i: (0, i))],
        out_specs=pl.BlockSpec((K, BA), lambda i: (0, i)),
        out_shape=jax.ShapeDtypeStruct((K, N), f32),
    )(Wk, xT)

    sc = _make_sc_gather(K, N, NP)
    s0, s1 = sc(gidx, yt.reshape(-1))

    s0t = s0.reshape(1, NP)
    s1t = s1.reshape(1, NP)
    BB = 2048
    gb = (N + BB - 1) // BB
    outT = pl.pallas_call(
        _gate_body,
        grid=(gb,),
        in_specs=[pl.BlockSpec((C, BB), lambda i: (0, i)),
                  pl.BlockSpec((1, BB), lambda i: (0, i)),
                  pl.BlockSpec((1, BB), lambda i: (0, i))],
        out_specs=pl.BlockSpec((C, BB), lambda i: (0, i)),
        out_shape=jax.ShapeDtypeStruct((C, N), f32),
    )(xT, s0t, s1t)
    return outT.T
